# baseline (speedup 1.0000x reference)
import functools

import jax
import jax.numpy as jnp
from jax import lax
from jax.experimental import pallas as pl
from jax.experimental.pallas import tpu as pltpu
from jax.experimental.pallas import tpu_sc as plsc

_CHUNK_ROWS = 64
_NBUF = 2


def kernel(x, emb_table):
    L = x.shape[1]
    D = emb_table.shape[1]

    info = plsc.get_sparse_core_info()
    num_workers = info.num_cores * info.num_subcores
    rows_per_worker = L // num_workers
    n_chunks = rows_per_worker // _CHUNK_ROWS

    mesh = plsc.VectorSubcoreMesh(core_axis_name="c", subcore_axis_name="s")

    @functools.partial(
        pl.kernel,
        mesh=mesh,
        out_type=jax.ShapeDtypeStruct((1, L, D), jnp.float32),
        scratch_types=(
            [
                pltpu.VMEM_SHARED((info.num_subcores, _CHUNK_ROWS, D), jnp.float32)
                for _ in range(_NBUF)
            ]
            + [pltpu.SemaphoreType.DMA for _ in range(2 * _NBUF)]
        ),
    )
    def copy_rows(table_hbm, out_hbm, *scratch):
        bufs = scratch[:_NBUF]
        sin = scratch[_NBUF : 2 * _NBUF]
        sout = scratch[2 * _NBUF :]
        sid = lax.axis_index("s")
        wid = sid * info.num_cores + lax.axis_index("c")
        base = wid * rows_per_worker

        def read(i):
            return pltpu.async_copy(
                table_hbm.at[pl.ds(base + i * _CHUNK_ROWS, _CHUNK_ROWS)],
                bufs[i % _NBUF].at[sid],
                sin[i % _NBUF],
            )

        def write(i):
            return pltpu.async_copy(
                bufs[i % _NBUF].at[sid],
                out_hbm.at[0, pl.ds(base + i * _CHUNK_ROWS, _CHUNK_ROWS)],
                sout[i % _NBUF],
            )

        in_dma = [None] * n_chunks
        out_dma = [None] * n_chunks
        for i in range(min(_NBUF, n_chunks)):
            in_dma[i] = read(i)
        for i in range(n_chunks):
            in_dma[i].wait()
            out_dma[i] = write(i)
            j = i + _NBUF
            if j < n_chunks:
                out_dma[j - _NBUF].wait()
                in_dma[j] = read(j)
        for i in range(max(n_chunks - _NBUF, 0), n_chunks):
            out_dma[i].wait()

    return copy_rows(emb_table)


# speedup vs baseline: 1.0216x; 1.0216x over previous
import functools

import jax
import jax.numpy as jnp
from jax import lax
from jax.experimental import pallas as pl
from jax.experimental.pallas import tpu as pltpu
from jax.experimental.pallas import tpu_sc as plsc

_R_ROWS = 32
_W_ROWS = 64
_RING_R = 4


def kernel(x, emb_table):
    L = x.shape[1]
    D = emb_table.shape[1]

    info = plsc.get_sparse_core_info()
    num_workers = info.num_cores * info.num_subcores
    rows_per_worker = L // num_workers
    n_reads = rows_per_worker // _R_ROWS
    n_writes = rows_per_worker // _W_ROWS
    ring_w = _RING_R * _R_ROWS // _W_ROWS

    mesh = plsc.VectorSubcoreMesh(core_axis_name="c", subcore_axis_name="s")

    @functools.partial(
        pl.kernel,
        mesh=mesh,
        out_type=jax.ShapeDtypeStruct((1, L, D), jnp.float32),
        scratch_types=(
            [pltpu.VMEM((_RING_R * _R_ROWS, D), jnp.float32)]
            + [pltpu.SemaphoreType.DMA for _ in range(_RING_R + ring_w)]
        ),
    )
    def copy_rows(table_hbm, out_hbm, ring, *sems):
        sin = sems[:_RING_R]
        sout = sems[_RING_R:]
        wid = lax.axis_index("s") * info.num_cores + lax.axis_index("c")
        base = wid * rows_per_worker

        def read(j):
            return pltpu.async_copy(
                table_hbm.at[pl.ds(base + j * _R_ROWS, _R_ROWS)],
                ring.at[pl.ds((j % _RING_R) * _R_ROWS, _R_ROWS)],
                sin[j % _RING_R],
            )

        def write(i):
            return pltpu.async_copy(
                ring.at[pl.ds((i % ring_w) * _W_ROWS, _W_ROWS)],
                out_hbm.at[0, pl.ds(base + i * _W_ROWS, _W_ROWS)],
                sout[i % ring_w],
            )

        rpw = _W_ROWS // _R_ROWS
        in_dma = [None] * n_reads
        out_dma = [None] * n_writes
        for j in range(min(_RING_R, n_reads)):
            in_dma[j] = read(j)
        for i in range(n_writes):
            for j in range(i * rpw, (i + 1) * rpw):
                in_dma[j].wait()
            out_dma[i] = write(i)
            for j in range(i * rpw + _RING_R, (i + 1) * rpw + _RING_R):
                if j < n_reads:
                    if j % rpw == 0:
                        out_dma[j // rpw - ring_w].wait()
                    in_dma[j] = read(j)
        for i in range(max(n_writes - ring_w, 0), n_writes):
            out_dma[i].wait()

    return copy_rows(emb_table)


# speedup vs baseline: 1.0595x; 1.0371x over previous
import functools

import jax
import jax.numpy as jnp
from jax import lax
from jax.experimental import pallas as pl
from jax.experimental.pallas import tpu as pltpu
from jax.experimental.pallas import tpu_sc as plsc

_CHUNK_ROWS = 32
_NBUF = 5


def kernel(x, emb_table):
    L = x.shape[1]
    D = emb_table.shape[1]

    info = plsc.get_sparse_core_info()
    num_workers = info.num_cores * info.num_subcores
    rows_per_worker = L // num_workers
    n_chunks = rows_per_worker // _CHUNK_ROWS

    mesh = plsc.VectorSubcoreMesh(core_axis_name="c", subcore_axis_name="s")

    @functools.partial(
        pl.kernel,
        mesh=mesh,
        out_type=jax.ShapeDtypeStruct((1, L, D), jnp.float32),
        scratch_types=(
            [pltpu.VMEM((_CHUNK_ROWS, D), jnp.float32) for _ in range(_NBUF)]
            + [pltpu.SemaphoreType.DMA for _ in range(_NBUF)]
        ),
    )
    def copy_rows(table_hbm, out_hbm, *scratch):
        bufs = scratch[:_NBUF]
        sin = scratch[_NBUF:]
        sout = sin
        wid = lax.axis_index("s") * info.num_cores + lax.axis_index("c")
        base = wid * rows_per_worker

        def read(i):
            return pltpu.async_copy(
                table_hbm.at[pl.ds(base + i * _CHUNK_ROWS, _CHUNK_ROWS)],
                bufs[i % _NBUF],
                sin[i % _NBUF],
            )

        def write(i):
            return pltpu.async_copy(
                bufs[i % _NBUF],
                out_hbm.at[0, pl.ds(base + i * _CHUNK_ROWS, _CHUNK_ROWS)],
                sout[i % _NBUF],
            )

        in_dma = [None] * n_chunks
        out_dma = [None] * n_chunks
        for i in range(min(_NBUF, n_chunks)):
            in_dma[i] = read(i)
        for i in range(n_chunks):
            in_dma[i].wait()
            out_dma[i] = write(i)
            j = i + _NBUF
            if j < n_chunks:
                out_dma[j - _NBUF].wait()
                in_dma[j] = read(j)
        for i in range(max(n_chunks - _NBUF, 0), n_chunks):
            out_dma[i].wait()

    return copy_rows(emb_table)
